# baseline (speedup 1.0000x reference)
import functools

import jax
import jax.numpy as jnp
from jax import lax
from jax.experimental import pallas as pl
from jax.experimental.pallas import tpu as pltpu
from jax.experimental.pallas import tpu_sc as plsc

_NC = 2
_NS = 16
_LANES = 16


def _transpose_tables(focal_t, context_t):
    D, V = focal_t.shape
    W = 4096
    grid = (V + W - 1) // W

    def body(f_ref, c_ref, fo_ref, co_ref):
        fo_ref[...] = f_ref[...].T
        co_ref[...] = c_ref[...].T

    out_shape = jax.ShapeDtypeStruct((V, D), jnp.float32)
    return pl.pallas_call(
        body,
        grid=(grid,),
        in_specs=[
            pl.BlockSpec((D, W), lambda i: (0, i)),
            pl.BlockSpec((D, W), lambda i: (0, i)),
        ],
        out_specs=[
            pl.BlockSpec((W, D), lambda i: (i, 0)),
            pl.BlockSpec((W, D), lambda i: (i, 0)),
        ],
        out_shape=(out_shape, out_shape),
    )(focal_t, context_t)


def kernel(tokens, focal_table, context_table):
    B, L = tokens.shape
    V, D = focal_table.shape
    N = B * L
    NW = _NC * _NS
    C = 512
    b_per_w = N // NW
    n_chunks = b_per_w // C
    assert b_per_w * NW == N and n_chunks * C == b_per_w and n_chunks % 2 == 0

    f_rm, c_rm = _transpose_tables(focal_table.T, context_table.T)

    idx = tokens.reshape(N).astype(jnp.int32)
    mesh = plsc.VectorSubcoreMesh(core_axis_name="c", subcore_axis_name="s")

    @functools.partial(
        pl.kernel,
        out_type=jax.ShapeDtypeStruct((N, D), jnp.float32),
        mesh=mesh,
        scratch_types=[
            pltpu.VMEM((b_per_w,), jnp.int32),
            pltpu.VMEM((C, D), jnp.float32),
            pltpu.VMEM((C, D), jnp.float32),
            pltpu.VMEM((C, D), jnp.float32),
            pltpu.VMEM((C, D), jnp.float32),
            pltpu.SemaphoreType.DMA,
            pltpu.SemaphoreType.DMA,
        ],
        compiler_params=pltpu.CompilerParams(use_tc_tiling_on_sc=False),
    )
    def sc_kernel(idx_hbm, focal_hbm, context_hbm, out_hbm,
                  idx_v, f0, c0, f1, c1, sem0, sem1):
        wid = lax.axis_index("s") * _NC + lax.axis_index("c")
        base = wid * b_per_w
        pltpu.sync_copy(idx_hbm.at[pl.ds(base, b_per_w)], idx_v)

        f_bufs, c_bufs, sems = (f0, f1), (c0, c1), (sem0, sem1)

        def issue(g, slot):
            sl = idx_v.at[pl.ds(g * C, C)]
            pltpu.async_copy(focal_hbm.at[sl], f_bufs[slot], sems[slot])
            pltpu.async_copy(context_hbm.at[sl], c_bufs[slot], sems[slot])

        def drain(g, slot):
            sl = idx_v.at[pl.ds(g * C, C)]
            pltpu.make_async_copy(focal_hbm.at[sl], f_bufs[slot],
                                  sems[slot]).wait()
            pltpu.make_async_copy(context_hbm.at[sl], c_bufs[slot],
                                  sems[slot]).wait()

        def process(g, slot):
            f_b, c_b = f_bufs[slot], c_bufs[slot]

            @plsc.parallel_loop(0, C, step=1, unroll=8)
            def _(r):
                plsc.addupdate(f_b.at[r, pl.ds(0, _LANES)],
                               c_b[r, pl.ds(0, _LANES)])
                plsc.addupdate(f_b.at[r, pl.ds(_LANES, _LANES)],
                               c_b[r, pl.ds(_LANES, _LANES)])

            pltpu.sync_copy(f_b, out_hbm.at[pl.ds(base + g * C, C)])

        issue(0, 0)
        issue(1, 1)

        @pl.loop(0, n_chunks, step=2)
        def _(g):
            drain(g, 0)
            process(g, 0)

            @pl.when(g + 2 < n_chunks)
            def _():
                issue(g + 2, 0)

            drain(g + 1, 1)
            process(g + 1, 1)

            @pl.when(g + 3 < n_chunks)
            def _():
                issue(g + 3, 1)

    out = sc_kernel(idx, f_rm, c_rm)
    return out.reshape(B, L, D)


# speedup vs baseline: 1.4040x; 1.4040x over previous
import functools

import jax
import jax.numpy as jnp
from jax import lax
from jax.experimental import pallas as pl
from jax.experimental.pallas import tpu as pltpu
from jax.experimental.pallas import tpu_sc as plsc

_NC = 2
_NS = 16
_LANES = 16


def kernel(tokens, focal_table, context_table):
    B, L = tokens.shape
    V, D = focal_table.shape
    NW = _NC * _NS
    CB = B // (128 * NW)
    W = 128 * CB
    assert W * NW == B and D == 32 and L % 2 == 0

    tokens_t = tokens.T
    mesh = plsc.VectorSubcoreMesh(core_axis_name="c", subcore_axis_name="s")

    @functools.partial(
        pl.kernel,
        out_type=jax.ShapeDtypeStruct((L, D // 8, B // 128, 8, 128),
                                      jnp.float32),
        mesh=mesh,
        scratch_types=[
            pltpu.VMEM((L, W), jnp.int32),
            pltpu.VMEM((W, D), jnp.float32),
            pltpu.VMEM((W, D), jnp.float32),
            pltpu.VMEM((W, D), jnp.float32),
            pltpu.VMEM((W, D), jnp.float32),
            pltpu.VMEM((D, 128), jnp.float32),
            pltpu.VMEM((D, 128), jnp.float32),
            pltpu.SemaphoreType.DMA,
            pltpu.SemaphoreType.DMA,
            pltpu.SemaphoreType.DMA,
            pltpu.SemaphoreType.DMA,
        ],
        compiler_params=pltpu.CompilerParams(use_tc_tiling_on_sc=False,
                                             needs_layout_passes=False),
    )
    def sc_kernel(tok_hbm, focal_hbm, context_hbm, out_hbm,
                  idx_v, f0, c0, f1, c1, t0, t1,
                  sem0, sem1, osem0, osem1):
        wid = lax.axis_index("s") * _NC + lax.axis_index("c")
        b0 = wid * W
        pltpu.sync_copy(tok_hbm.at[:, pl.ds(b0, W)], idx_v)

        f_bufs, c_bufs, sems = (f0, f1), (c0, c1), (sem0, sem1)
        tbufs, osems = (t0, t1), (osem0, osem1)
        iota = lax.iota(jnp.int32, _LANES)

        def issue(l, slot):
            sl = idx_v.at[l]
            pltpu.async_copy(focal_hbm.at[sl], f_bufs[slot], sems[slot])
            pltpu.async_copy(context_hbm.at[sl], c_bufs[slot], sems[slot])

        def drain(l, slot):
            sl = idx_v.at[l]
            pltpu.make_async_copy(focal_hbm.at[sl], f_bufs[slot],
                                  sems[slot]).wait()
            pltpu.make_async_copy(context_hbm.at[sl], c_bufs[slot],
                                  sems[slot]).wait()

        def drain_out(l, ts):
            for a in range(D // 8):
                pltpu.make_async_copy(
                    tbufs[ts].at[pl.ds(8 * a, 8)],
                    out_hbm.at[l, a, CB * wid],
                    osems[ts]).wait()

        def process(l, slot):
            f_b, c_b = f_bufs[slot], c_bufs[slot]
            for cp in range(CB):
                ts = cp % 2
                if cp < 2:
                    @pl.when(l > 0)
                    def _():
                        drain_out(l, ts)
                else:
                    drain_out(l, ts)
                t_b = tbufs[ts]
                for jg in range(8):
                    jidx = 128 * cp + 16 * jg + iota

                    @plsc.parallel_loop(0, D, step=1, unroll=4)
                    def _(d):
                        didx = jnp.full((_LANES,), 0, jnp.int32) + d
                        fv = plsc.load_gather(f_b, [jidx, didx])
                        cv = plsc.load_gather(c_b, [jidx, didx])
                        t_b[d, pl.ds(16 * jg, _LANES)] = fv + cv

                for a in range(D // 8):
                    pltpu.async_copy(
                        t_b.at[pl.ds(8 * a, 8)],
                        out_hbm.at[l, a, CB * wid + cp],
                        osems[ts])

        issue(0, 0)
        issue(1, 1)

        @pl.loop(0, L, step=2)
        def _(l):
            drain(l, 0)
            process(l, 0)

            @pl.when(l + 2 < L)
            def _():
                issue(l + 2, 0)

            drain(l + 1, 1)
            process(l + 1, 1)

            @pl.when(l + 3 < L)
            def _():
                issue(l + 3, 1)

        drain_out(L - 1, 0)
        drain_out(L - 1, 1)

    out5 = sc_kernel(tokens_t, focal_table, context_table)
    return out5.transpose(2, 4, 0, 1, 3).reshape(B, L, D)


# speedup vs baseline: 1.4551x; 1.0364x over previous
import functools

import jax
import jax.numpy as jnp
from jax import lax
from jax.experimental import pallas as pl
from jax.experimental.pallas import tpu as pltpu
from jax.experimental.pallas import tpu_sc as plsc

_NC = 2
_NS = 16
_LANES = 16


def _repack_tables(focal_t, context_t, tail_f, tail_c):
    D, V = focal_t.shape
    n_full = V // 128
    tail = V - n_full * 128
    NW = _NC * _NS
    K = -(-n_full // NW)
    if K % 2:
        K += 1

    mesh = plsc.VectorSubcoreMesh(core_axis_name="c", subcore_axis_name="s")
    out_t = jax.ShapeDtypeStruct((V // 4, 128), jnp.float32)

    @functools.partial(
        pl.kernel,
        out_type=(out_t, out_t),
        mesh=mesh,
        scratch_types=[
            pltpu.VMEM((D, 128), jnp.float32),
            pltpu.VMEM((D, 128), jnp.float32),
            pltpu.VMEM((128 // 4, 128), jnp.float32),
            pltpu.VMEM((128 // 4, 128), jnp.float32),
            pltpu.SemaphoreType.DMA,
            pltpu.SemaphoreType.DMA,
            pltpu.SemaphoreType.DMA,
            pltpu.SemaphoreType.DMA,
        ],
        compiler_params=pltpu.CompilerParams(use_tc_tiling_on_sc=True,
                                             needs_layout_passes=False),
    )
    def rk(f_hbm, c_hbm, tf_hbm, tc_hbm, fo_hbm, co_hbm,
           b0, b1, t0, t1, si0, si1, so0, so1):
        wid = lax.axis_index("s") * _NC + lax.axis_index("c")
        bufs, tbufs = (b0, b1), (t0, t1)
        sis, sos = (si0, si1), (so0, so1)
        iota = lax.iota(jnp.int32, _LANES)
        d_half = (iota, iota + _LANES)

        def transpose_block(slot, n_j):
            buf, tbuf = bufs[slot], tbufs[slot]

            @plsc.parallel_loop(0, n_j, step=1, unroll=4)
            def _(j):
                jidx = jnp.full((_LANES,), 0, jnp.int32) + j
                r = j // 4
                q0 = (j % 4) * 32
                for h in range(2):
                    v = plsc.load_gather(buf, [d_half[h], jidx])
                    tbuf[r, pl.ds(q0 + 16 * h, _LANES)] = v

        for src, tsrc, dst in ((f_hbm, tf_hbm, fo_hbm),
                               (c_hbm, tc_hbm, co_hbm)):
            def issue(c, slot):
                pltpu.async_copy(
                    src.at[pl.ds(0, D), pl.ds(c * 128, 128)],
                    bufs[slot], sis[slot])

            def drain_in(slot):
                pltpu.make_async_copy(
                    src.at[pl.ds(0, D), pl.ds(0, 128)],
                    bufs[slot], sis[slot]).wait()

            def issue_out(c, slot):
                pltpu.async_copy(tbufs[slot], dst.at[pl.ds(32 * c, 32)],
                                 sos[slot])

            def drain_out(slot):
                pltpu.make_async_copy(tbufs[slot],
                                      dst.at[pl.ds(0, 32)],
                                      sos[slot]).wait()

            def step(k, slot):
                c = wid + NW * k

                @pl.when(c < n_full)
                def _():
                    drain_in(slot)

                    @pl.when(k >= 2)
                    def _():
                        drain_out(slot)

                    transpose_block(slot, 128)
                    issue_out(c, slot)

                    @pl.when(c + 2 * NW < n_full)
                    def _():
                        issue(c + 2 * NW, slot)

            @pl.when(wid < n_full)
            def _():
                issue(wid, 0)

            @pl.when(wid + NW < n_full)
            def _():
                issue(wid + NW, 1)

            @pl.loop(0, K, step=2)
            def _(k):
                step(k, 0)
                step(k + 1, 1)

            drain_out(0)
            drain_out(1)

            if tail:
                @pl.when(wid == 0)
                def _():
                    pltpu.sync_copy(tsrc, bufs[0].at[pl.ds(0, tail // 4)])
                    pltpu.sync_copy(bufs[0].at[pl.ds(0, tail // 4)],
                                    dst.at[pl.ds(32 * n_full, tail // 4)])

    return rk(focal_t, context_t, tail_f, tail_c)


def kernel(tokens, focal_table, context_table):
    B, L = tokens.shape
    V, D = focal_table.shape
    NW = _NC * _NS
    CB = B // (128 * NW)
    W = 128 * CB
    assert W * NW == B and D == 32 and L % 2 == 0

    tokens_t = tokens.T
    n_full128 = (V // 128) * 128
    tail_f = focal_table[n_full128:].reshape(-1, 128)
    tail_c = context_table[n_full128:].reshape(-1, 128)
    f128, c128 = _repack_tables(focal_table.T, context_table.T,
                                tail_f, tail_c)
    f_rm = f128.reshape(V, D)
    c_rm = c128.reshape(V, D)
    mesh = plsc.VectorSubcoreMesh(core_axis_name="c", subcore_axis_name="s")

    @functools.partial(
        pl.kernel,
        out_type=jax.ShapeDtypeStruct((L, D // 8, B // 128, 8, 128),
                                      jnp.float32),
        mesh=mesh,
        scratch_types=[
            pltpu.VMEM((L, W), jnp.int32),
            pltpu.VMEM((W, D), jnp.float32),
            pltpu.VMEM((W, D), jnp.float32),
            pltpu.VMEM((W, D), jnp.float32),
            pltpu.VMEM((W, D), jnp.float32),
            pltpu.VMEM((D, 128), jnp.float32),
            pltpu.VMEM((D, 128), jnp.float32),
            pltpu.SemaphoreType.DMA,
            pltpu.SemaphoreType.DMA,
            pltpu.SemaphoreType.DMA,
            pltpu.SemaphoreType.DMA,
        ],
        compiler_params=pltpu.CompilerParams(use_tc_tiling_on_sc=False,
                                             needs_layout_passes=False),
    )
    def sc_kernel(tok_hbm, focal_hbm, context_hbm, out_hbm,
                  idx_v, f0, c0, f1, c1, t0, t1,
                  sem0, sem1, osem0, osem1):
        wid = lax.axis_index("s") * _NC + lax.axis_index("c")
        b0 = wid * W
        pltpu.sync_copy(tok_hbm.at[:, pl.ds(b0, W)], idx_v)

        f_bufs, c_bufs, sems = (f0, f1), (c0, c1), (sem0, sem1)
        tbufs, osems = (t0, t1), (osem0, osem1)
        iota = lax.iota(jnp.int32, _LANES)

        def issue(l, slot):
            sl = idx_v.at[l]
            pltpu.async_copy(focal_hbm.at[sl], f_bufs[slot], sems[slot])
            pltpu.async_copy(context_hbm.at[sl], c_bufs[slot], sems[slot])

        def drain(l, slot):
            sl = idx_v.at[l]
            pltpu.make_async_copy(focal_hbm.at[sl], f_bufs[slot],
                                  sems[slot]).wait()
            pltpu.make_async_copy(context_hbm.at[sl], c_bufs[slot],
                                  sems[slot]).wait()

        def drain_out(l, ts):
            for a in range(D // 8):
                pltpu.make_async_copy(
                    tbufs[ts].at[pl.ds(8 * a, 8)],
                    out_hbm.at[l, a, CB * wid],
                    osems[ts]).wait()

        def process(l, slot):
            f_b, c_b = f_bufs[slot], c_bufs[slot]
            for cp in range(CB):
                ts = cp % 2
                if cp < 2:
                    @pl.when(l > 0)
                    def _():
                        drain_out(l, ts)
                else:
                    drain_out(l, ts)
                t_b = tbufs[ts]
                for jg in range(8):
                    jidx = 128 * cp + 16 * jg + iota

                    @plsc.parallel_loop(0, D, step=1, unroll=4)
                    def _(d):
                        didx = jnp.full((_LANES,), 0, jnp.int32) + d
                        fv = plsc.load_gather(f_b, [jidx, didx])
                        cv = plsc.load_gather(c_b, [jidx, didx])
                        t_b[d, pl.ds(16 * jg, _LANES)] = fv + cv

                for a in range(D // 8):
                    pltpu.async_copy(
                        t_b.at[pl.ds(8 * a, 8)],
                        out_hbm.at[l, a, CB * wid + cp],
                        osems[ts])

        issue(0, 0)
        issue(1, 1)

        @pl.loop(0, L, step=2)
        def _(l):
            drain(l, 0)
            process(l, 0)

            @pl.when(l + 2 < L)
            def _():
                issue(l + 2, 0)

            drain(l + 1, 1)
            process(l + 1, 1)

            @pl.when(l + 3 < L)
            def _():
                issue(l + 3, 1)

        drain_out(L - 1, 0)
        drain_out(L - 1, 1)

    out5 = sc_kernel(tokens_t, f_rm, c_rm)
    return out5.transpose(2, 4, 0, 1, 3).reshape(B, L, D)


# speedup vs baseline: 2.5247x; 1.7350x over previous
import functools

import jax
import jax.numpy as jnp
from jax import lax
from jax.experimental import pallas as pl
from jax.experimental.pallas import tpu as pltpu
from jax.experimental.pallas import tpu_sc as plsc

_NC = 2
_NS = 16
_LANES = 16


def _repack_tables(focal_t, context_t, tail_f, tail_c):
    D, V = focal_t.shape
    n_full = V // 128
    tail = V - n_full * 128
    NW = _NC * _NS
    K = -(-n_full // NW)
    if K % 2:
        K += 1

    mesh = plsc.VectorSubcoreMesh(core_axis_name="c", subcore_axis_name="s")
    out_t = jax.ShapeDtypeStruct((V // 4, 128), jnp.float32)

    @functools.partial(
        pl.kernel,
        out_type=(out_t, out_t),
        mesh=mesh,
        scratch_types=[
            pltpu.VMEM((D, 129), jnp.float32),
            pltpu.VMEM((D, 129), jnp.float32),
            pltpu.VMEM((128 // 4, 128), jnp.float32),
            pltpu.VMEM((128 // 4, 128), jnp.float32),
            pltpu.SemaphoreType.DMA,
            pltpu.SemaphoreType.DMA,
            pltpu.SemaphoreType.DMA,
            pltpu.SemaphoreType.DMA,
        ],
        compiler_params=pltpu.CompilerParams(use_tc_tiling_on_sc=True,
                                             needs_layout_passes=False),
    )
    def rk(f_hbm, c_hbm, tf_hbm, tc_hbm, fo_hbm, co_hbm,
           b0, b1, t0, t1, si0, si1, so0, so1):
        wid = lax.axis_index("s") * _NC + lax.axis_index("c")
        bufs, tbufs = (b0, b1), (t0, t1)
        sis, sos = (si0, si1), (so0, so1)
        iota = lax.iota(jnp.int32, _LANES)
        d_half = (iota, iota + _LANES)

        def transpose_block(slot, n_j):
            buf, tbuf = bufs[slot], tbufs[slot]

            @plsc.parallel_loop(0, n_j, step=1, unroll=4)
            def _(j):
                jidx = jnp.full((_LANES,), 0, jnp.int32) + j
                r = j // 4
                q0 = (j % 4) * 32
                for h in range(2):
                    v = plsc.load_gather(buf, [d_half[h], jidx])
                    tbuf[r, pl.ds(q0 + 16 * h, _LANES)] = v

        for src, tsrc, dst in ((f_hbm, tf_hbm, fo_hbm),
                               (c_hbm, tc_hbm, co_hbm)):
            def issue(c, slot):
                pltpu.async_copy(
                    src.at[pl.ds(0, D), pl.ds(c * 128, 128)],
                    bufs[slot].at[:, pl.ds(0, 128)], sis[slot])

            def drain_in(slot):
                pltpu.make_async_copy(
                    src.at[pl.ds(0, D), pl.ds(0, 128)],
                    bufs[slot].at[:, pl.ds(0, 128)], sis[slot]).wait()

            def issue_out(c, slot):
                pltpu.async_copy(tbufs[slot], dst.at[pl.ds(32 * c, 32)],
                                 sos[slot])

            def drain_out(slot):
                pltpu.make_async_copy(tbufs[slot],
                                      dst.at[pl.ds(0, 32)],
                                      sos[slot]).wait()

            def step(k, slot):
                c = wid + NW * k

                @pl.when(c < n_full)
                def _():
                    drain_in(slot)

                    @pl.when(k >= 2)
                    def _():
                        drain_out(slot)

                    transpose_block(slot, 128)
                    issue_out(c, slot)

                    @pl.when(c + 2 * NW < n_full)
                    def _():
                        issue(c + 2 * NW, slot)

            @pl.when(wid < n_full)
            def _():
                issue(wid, 0)

            @pl.when(wid + NW < n_full)
            def _():
                issue(wid + NW, 1)

            @pl.loop(0, K, step=2)
            def _(k):
                step(k, 0)
                step(k + 1, 1)

            drain_out(0)
            drain_out(1)

            if tail:
                @pl.when(wid == 0)
                def _():
                    pltpu.sync_copy(tsrc,
                                    tbufs[0].at[pl.ds(0, tail // 4)])
                    pltpu.sync_copy(tbufs[0].at[pl.ds(0, tail // 4)],
                                    dst.at[pl.ds(32 * n_full, tail // 4)])

    return rk(focal_t, context_t, tail_f, tail_c)


def kernel(tokens, focal_table, context_table):
    B, L = tokens.shape
    V, D = focal_table.shape
    NW = _NC * _NS
    CB = B // (128 * NW)
    W = 128 * CB
    assert W * NW == B and D == 32 and L % 2 == 0

    tokens_t = tokens.T
    n_full128 = (V // 128) * 128
    tail_f = focal_table[n_full128:].reshape(-1, 128)
    tail_c = context_table[n_full128:].reshape(-1, 128)
    f128, c128 = _repack_tables(focal_table.T, context_table.T,
                                tail_f, tail_c)
    f_rm = f128.reshape(V, D)
    c_rm = c128.reshape(V, D)
    mesh = plsc.VectorSubcoreMesh(core_axis_name="c", subcore_axis_name="s")

    @functools.partial(
        pl.kernel,
        out_type=jax.ShapeDtypeStruct((L, D // 8, B // 128, 8, 128),
                                      jnp.float32),
        mesh=mesh,
        scratch_types=[
            pltpu.VMEM((L, W), jnp.int32),
            pltpu.VMEM((W, D), jnp.float32),
            pltpu.VMEM((W, D), jnp.float32),
            pltpu.VMEM((W, D), jnp.float32),
            pltpu.VMEM((W, D), jnp.float32),
            pltpu.VMEM((D, 129), jnp.float32),
            pltpu.VMEM((D, 129), jnp.float32),
            pltpu.SemaphoreType.DMA,
            pltpu.SemaphoreType.DMA,
            pltpu.SemaphoreType.DMA,
            pltpu.SemaphoreType.DMA,
        ],
        compiler_params=pltpu.CompilerParams(use_tc_tiling_on_sc=False,
                                             needs_layout_passes=False),
    )
    def sc_kernel(tok_hbm, focal_hbm, context_hbm, out_hbm,
                  idx_v, f0, c0, f1, c1, t0, t1,
                  sem0, sem1, osem0, osem1):
        wid = lax.axis_index("s") * _NC + lax.axis_index("c")
        b0 = wid * W
        pltpu.sync_copy(tok_hbm.at[:, pl.ds(b0, W)], idx_v)

        f_bufs, c_bufs, sems = (f0, f1), (c0, c1), (sem0, sem1)
        tbufs, osems = (t0, t1), (osem0, osem1)
        iota = lax.iota(jnp.int32, _LANES)

        def issue(l, slot):
            sl = idx_v.at[l]
            pltpu.async_copy(focal_hbm.at[sl], f_bufs[slot], sems[slot])
            pltpu.async_copy(context_hbm.at[sl], c_bufs[slot], sems[slot])

        def drain(l, slot):
            sl = idx_v.at[l]
            pltpu.make_async_copy(focal_hbm.at[sl], f_bufs[slot],
                                  sems[slot]).wait()
            pltpu.make_async_copy(context_hbm.at[sl], c_bufs[slot],
                                  sems[slot]).wait()

        d_half = (iota, iota + _LANES)

        def drain_out(l, ts):
            for a in range(D // 8):
                pltpu.make_async_copy(
                    tbufs[ts].at[pl.ds(8 * a, 8), pl.ds(0, 128)],
                    out_hbm.at[l, a, CB * wid],
                    osems[ts]).wait()

        def process(l, slot):
            f_b, c_b = f_bufs[slot], c_bufs[slot]
            for cp in range(CB):
                ts = cp % 2
                if cp < 2:
                    @pl.when(l > 0)
                    def _():
                        drain_out(l, ts)
                else:
                    drain_out(l, ts)
                t_b = tbufs[ts]

                @plsc.parallel_loop(0, 128, step=1, unroll=4)
                def _(j):
                    jj = 128 * cp + j
                    jidx = jnp.full((_LANES,), 0, jnp.int32) + j
                    for h in range(2):
                        v = (f_b[jj, pl.ds(16 * h, _LANES)]
                             + c_b[jj, pl.ds(16 * h, _LANES)])
                        plsc.store_scatter(t_b, [d_half[h], jidx], v)

                for a in range(D // 8):
                    pltpu.async_copy(
                        t_b.at[pl.ds(8 * a, 8), pl.ds(0, 128)],
                        out_hbm.at[l, a, CB * wid + cp],
                        osems[ts])

        issue(0, 0)
        issue(1, 1)

        @pl.loop(0, L, step=2)
        def _(l):
            drain(l, 0)
            process(l, 0)

            @pl.when(l + 2 < L)
            def _():
                issue(l + 2, 0)

            drain(l + 1, 1)
            process(l + 1, 1)

            @pl.when(l + 3 < L)
            def _():
                issue(l + 3, 1)

        drain_out(L - 1, 0)
        drain_out(L - 1, 1)

    out5 = sc_kernel(tokens_t, f_rm, c_rm)
    return out5.transpose(2, 4, 0, 1, 3).reshape(B, L, D)


# speedup vs baseline: 2.5269x; 1.0009x over previous
import functools

import jax
import jax.numpy as jnp
from jax import lax
from jax.experimental import pallas as pl
from jax.experimental.pallas import tpu as pltpu
from jax.experimental.pallas import tpu_sc as plsc

_NC = 2
_NS = 16
_LANES = 16


def _repack_tables(focal_t, context_t, tail_f, tail_c):
    D, V = focal_t.shape
    n_full = V // 128
    tail = V - n_full * 128
    NW = _NC * _NS
    K = -(-n_full // NW)
    if K % 2:
        K += 1

    mesh = plsc.VectorSubcoreMesh(core_axis_name="c", subcore_axis_name="s")
    out_t = jax.ShapeDtypeStruct((V // 4, 128), jnp.float32)

    @functools.partial(
        pl.kernel,
        out_type=(out_t, out_t),
        mesh=mesh,
        scratch_types=[
            pltpu.VMEM((D, 129), jnp.float32),
            pltpu.VMEM((D, 129), jnp.float32),
            pltpu.VMEM((128 // 4, 128), jnp.float32),
            pltpu.VMEM((128 // 4, 128), jnp.float32),
            pltpu.SemaphoreType.DMA,
            pltpu.SemaphoreType.DMA,
            pltpu.SemaphoreType.DMA,
            pltpu.SemaphoreType.DMA,
        ],
        compiler_params=pltpu.CompilerParams(use_tc_tiling_on_sc=True,
                                             needs_layout_passes=False),
    )
    def rk(f_hbm, c_hbm, tf_hbm, tc_hbm, fo_hbm, co_hbm,
           b0, b1, t0, t1, si0, si1, so0, so1):
        wid = lax.axis_index("s") * _NC + lax.axis_index("c")
        bufs, tbufs = (b0, b1), (t0, t1)
        sis, sos = (si0, si1), (so0, so1)
        iota = lax.iota(jnp.int32, _LANES)
        d_half = (iota, iota + _LANES)

        def transpose_block(slot, n_j):
            buf, tbuf = bufs[slot], tbufs[slot]

            @plsc.parallel_loop(0, n_j, step=1, unroll=8)
            def _(j):
                jidx = jnp.full((_LANES,), 0, jnp.int32) + j
                r = j // 4
                q0 = (j % 4) * 32
                for h in range(2):
                    v = plsc.load_gather(buf, [d_half[h], jidx])
                    tbuf[r, pl.ds(q0 + 16 * h, _LANES)] = v

        for src, tsrc, dst in ((f_hbm, tf_hbm, fo_hbm),
                               (c_hbm, tc_hbm, co_hbm)):
            def issue(c, slot):
                pltpu.async_copy(
                    src.at[pl.ds(0, D), pl.ds(c * 128, 128)],
                    bufs[slot].at[:, pl.ds(0, 128)], sis[slot])

            def drain_in(slot):
                pltpu.make_async_copy(
                    src.at[pl.ds(0, D), pl.ds(0, 128)],
                    bufs[slot].at[:, pl.ds(0, 128)], sis[slot]).wait()

            def issue_out(c, slot):
                pltpu.async_copy(tbufs[slot], dst.at[pl.ds(32 * c, 32)],
                                 sos[slot])

            def drain_out(slot):
                pltpu.make_async_copy(tbufs[slot],
                                      dst.at[pl.ds(0, 32)],
                                      sos[slot]).wait()

            def step(k, slot):
                c = wid + NW * k

                @pl.when(c < n_full)
                def _():
                    drain_in(slot)

                    @pl.when(k >= 2)
                    def _():
                        drain_out(slot)

                    transpose_block(slot, 128)
                    issue_out(c, slot)

                    @pl.when(c + 2 * NW < n_full)
                    def _():
                        issue(c + 2 * NW, slot)

            @pl.when(wid < n_full)
            def _():
                issue(wid, 0)

            @pl.when(wid + NW < n_full)
            def _():
                issue(wid + NW, 1)

            @pl.loop(0, K, step=2)
            def _(k):
                step(k, 0)
                step(k + 1, 1)

            drain_out(0)
            drain_out(1)

            if tail:
                @pl.when(wid == 0)
                def _():
                    pltpu.sync_copy(tsrc,
                                    tbufs[0].at[pl.ds(0, tail // 4)])
                    pltpu.sync_copy(tbufs[0].at[pl.ds(0, tail // 4)],
                                    dst.at[pl.ds(32 * n_full, tail // 4)])

    return rk(focal_t, context_t, tail_f, tail_c)


def kernel(tokens, focal_table, context_table):
    B, L = tokens.shape
    V, D = focal_table.shape
    NW = _NC * _NS
    CB = B // (128 * NW)
    W = 128 * CB
    assert W * NW == B and D == 32 and L % 2 == 0

    tokens_t = tokens.T
    n_full128 = (V // 128) * 128
    tail_f = focal_table[n_full128:].reshape(-1, 128)
    tail_c = context_table[n_full128:].reshape(-1, 128)
    f128, c128 = _repack_tables(focal_table.T, context_table.T,
                                tail_f, tail_c)
    f_rm = f128.reshape(V, D)
    c_rm = c128.reshape(V, D)
    mesh = plsc.VectorSubcoreMesh(core_axis_name="c", subcore_axis_name="s")

    @functools.partial(
        pl.kernel,
        out_type=jax.ShapeDtypeStruct((L, D // 8, B // 128, 8, 128),
                                      jnp.float32),
        mesh=mesh,
        scratch_types=[
            pltpu.VMEM((L, W), jnp.int32),
            pltpu.VMEM((W, D), jnp.float32),
            pltpu.VMEM((W, D), jnp.float32),
            pltpu.VMEM((W, D), jnp.float32),
            pltpu.VMEM((W, D), jnp.float32),
            pltpu.VMEM((D, 129), jnp.float32),
            pltpu.VMEM((D, 129), jnp.float32),
            pltpu.SemaphoreType.DMA,
            pltpu.SemaphoreType.DMA,
            pltpu.SemaphoreType.DMA,
            pltpu.SemaphoreType.DMA,
        ],
        compiler_params=pltpu.CompilerParams(use_tc_tiling_on_sc=False,
                                             needs_layout_passes=False),
    )
    def sc_kernel(tok_hbm, focal_hbm, context_hbm, out_hbm,
                  idx_v, f0, c0, f1, c1, t0, t1,
                  sem0, sem1, osem0, osem1):
        wid = lax.axis_index("s") * _NC + lax.axis_index("c")
        b0 = wid * W
        pltpu.sync_copy(tok_hbm.at[:, pl.ds(b0, W)], idx_v)

        f_bufs, c_bufs, sems = (f0, f1), (c0, c1), (sem0, sem1)
        tbufs, osems = (t0, t1), (osem0, osem1)
        iota = lax.iota(jnp.int32, _LANES)

        def issue(l, slot):
            sl = idx_v.at[l]
            pltpu.async_copy(focal_hbm.at[sl], f_bufs[slot], sems[slot])
            pltpu.async_copy(context_hbm.at[sl], c_bufs[slot], sems[slot])

        def drain(l, slot):
            sl = idx_v.at[l]
            pltpu.make_async_copy(focal_hbm.at[sl], f_bufs[slot],
                                  sems[slot]).wait()
            pltpu.make_async_copy(context_hbm.at[sl], c_bufs[slot],
                                  sems[slot]).wait()

        d_half = (iota, iota + _LANES)

        def drain_out(l, ts):
            for a in range(D // 8):
                pltpu.make_async_copy(
                    tbufs[ts].at[pl.ds(8 * a, 8), pl.ds(0, 128)],
                    out_hbm.at[l, a, CB * wid],
                    osems[ts]).wait()

        def process(l, slot):
            f_b, c_b = f_bufs[slot], c_bufs[slot]
            for cp in range(CB):
                ts = cp % 2
                if cp < 2:
                    @pl.when(l > 0)
                    def _():
                        drain_out(l, ts)
                else:
                    drain_out(l, ts)
                t_b = tbufs[ts]

                @plsc.parallel_loop(0, 128, step=1, unroll=4)
                def _(j):
                    jj = 128 * cp + j
                    jidx = jnp.full((_LANES,), 0, jnp.int32) + j
                    for h in range(2):
                        v = (f_b[jj, pl.ds(16 * h, _LANES)]
                             + c_b[jj, pl.ds(16 * h, _LANES)])
                        plsc.store_scatter(t_b, [d_half[h], jidx], v)

                for a in range(D // 8):
                    pltpu.async_copy(
                        t_b.at[pl.ds(8 * a, 8), pl.ds(0, 128)],
                        out_hbm.at[l, a, CB * wid + cp],
                        osems[ts])

        issue(0, 0)
        issue(1, 1)

        @pl.loop(0, L, step=2)
        def _(l):
            drain(l, 0)
            process(l, 0)

            @pl.when(l + 2 < L)
            def _():
                issue(l + 2, 0)

            drain(l + 1, 1)
            process(l + 1, 1)

            @pl.when(l + 3 < L)
            def _():
                issue(l + 3, 1)

        drain_out(L - 1, 0)
        drain_out(L - 1, 1)

    out5 = sc_kernel(tokens_t, f_rm, c_rm)
    return out5.transpose(2, 4, 0, 1, 3).reshape(B, L, D)
